# Initial kernel scaffold; baseline (speedup 1.0000x reference)
import jax
import jax.numpy as jnp
from jax.experimental import pallas as pl


def kernel(tgt, seed_mask, pred_boxes, W1, b1, W2, b2, W3, b3, W4, b4, W5, b5):
    raise NotImplementedError("write your pallas kernel here")


# speedup vs baseline: 2.4099x; 2.4099x over previous
import numpy as np
import jax
import jax.numpy as jnp
from jax.experimental import pallas as pl

IOU_THR = 0.5
TOP_K = 10
NPF = 128


def _body(boxes_i_ref, boxes_ref, boxesT_ref, seed_col_ref, seed_row_ref,
          tgt_ref, w1t_ref, b1_ref, w2t_ref, b2_ref, w3wt_ref, s3_ref,
          b3_ref, w4t_ref, b4_ref, w5t_ref, b5_ref, ang_ref,
          out_ref, mask_ref):
    f32 = jnp.float32
    T = boxes_i_ref.shape[1]
    N = boxesT_ref.shape[2]
    D = tgt_ref.shape[2]

    bi = boxes_i_ref[0]
    bT = boxesT_ref[0]

    cx_i, cy_i, w_i, h_i = bi[:, 0:1], bi[:, 1:2], bi[:, 2:3], bi[:, 3:4]
    cx_j, cy_j, w_j, h_j = bT[0:1, :], bT[1:2, :], bT[2:3, :], bT[3:4, :]

    x0_i = cx_i - 0.5 * w_i
    x1_i = cx_i + 0.5 * w_i
    y0_i = cy_i - 0.5 * h_i
    y1_i = cy_i + 0.5 * h_i
    x0_j = cx_j - 0.5 * w_j
    x1_j = cx_j + 0.5 * w_j
    y0_j = cy_j - 0.5 * h_j
    y1_j = cy_j + 0.5 * h_j

    iw = jnp.maximum(jnp.minimum(x1_i, x1_j) - jnp.maximum(x0_i, x0_j), 0.0)
    ih = jnp.maximum(jnp.minimum(y1_i, y1_j) - jnp.maximum(y0_i, y0_j), 0.0)
    inter = iw * ih
    area_i = (x1_i - x0_i) * (y1_i - y0_i)
    area_j = (x1_j - x0_j) * (y1_j - y0_j)
    union = area_i + area_j - inter
    iou = inter / jnp.maximum(union, 1e-9)

    mask_ref[0] = iou >= IOU_THR

    neg_col = 1.0 - seed_col_ref[0]
    ov = iou * seed_row_ref[0] * neg_col

    iota = jax.lax.broadcasted_iota(jnp.int32, (T, N), 1)
    ang = ang_ref[...]
    boxes_all = boxes_ref[0]

    macc = jnp.full((T, D), -jnp.inf, dtype=f32)
    for _ in range(TOP_K):
        m = jnp.max(ov, axis=1, keepdims=True)
        idx = jnp.min(jnp.where(ov == m, iota, N), axis=1, keepdims=True)
        onehot = iota == idx
        nb = jnp.dot(onehot.astype(f32), boxes_all,
                     preferred_element_type=f32, precision=jax.lax.Precision.HIGHEST)
        ov = jnp.where(onehot, -1.0, ov)
        mk = (m >= IOU_THR).astype(f32)
        v = m * mk
        x = jnp.log(jnp.maximum(jnp.abs(nb - bi), 1e-7))
        waves = jnp.concatenate(
            [jnp.sin(x[:, 0:1] * ang), jnp.cos(x[:, 1:2] * ang),
             jnp.sin(x[:, 2:3] * ang), jnp.cos(x[:, 3:4] * ang)], axis=1)
        h = jnp.maximum(
            jnp.dot(waves, w3wt_ref[...], preferred_element_type=f32, precision=jax.lax.Precision.HIGHEST)
            + v * s3_ref[...] + b3_ref[...], 0.0)
        fk = jnp.dot(h, w4t_ref[...], preferred_element_type=f32, precision=jax.lax.Precision.HIGHEST) + b4_ref[...]
        macc = jnp.maximum(macc, fk * mk)

    h1 = jnp.maximum(
        jnp.dot(tgt_ref[0], w1t_ref[...], preferred_element_type=f32, precision=jax.lax.Precision.HIGHEST)
        + b1_ref[...], 0.0)
    cur = jnp.dot(h1, w2t_ref[...], preferred_element_type=f32, precision=jax.lax.Precision.HIGHEST) + b2_ref[...]
    pre = cur * neg_col + macc
    out = jnp.maximum(
        jnp.dot(pre, w5t_ref[...], preferred_element_type=f32, precision=jax.lax.Precision.HIGHEST)
        + b5_ref[...], 0.0) * neg_col
    out_ref[0] = out


def kernel(tgt, seed_mask, pred_boxes, W1, b1, W2, b2, W3, b3, W4, b4, W5, b5):
    bs, N, D = tgt.shape
    T = 200

    boxesT = jnp.transpose(pred_boxes, (0, 2, 1))
    seed_row = jnp.transpose(seed_mask, (0, 2, 1))

    s3 = jnp.sum(W3[:, :64], axis=1)[None, :]
    w3wt = jnp.transpose(W3[:, 64:])

    dim_t = 10000.0 ** (2.0 * np.floor(np.arange(NPF) / 2.0) / NPF)
    ang = jnp.asarray((2.0 * np.pi) / dim_t, jnp.float32)[None, :]

    row = lambda b: b[None, :]

    def const(shape):
        return pl.BlockSpec(shape, lambda b, i: (0,) * len(shape))

    out, mask = pl.pallas_call(
        _body,
        grid=(bs, N // T),
        in_specs=[
            pl.BlockSpec((1, T, 4), lambda b, i: (b, i, 0)),
            pl.BlockSpec((1, N, 4), lambda b, i: (b, 0, 0)),
            pl.BlockSpec((1, 4, N), lambda b, i: (b, 0, 0)),
            pl.BlockSpec((1, T, 1), lambda b, i: (b, i, 0)),
            pl.BlockSpec((1, 1, N), lambda b, i: (b, 0, 0)),
            pl.BlockSpec((1, T, D), lambda b, i: (b, i, 0)),
            const((D, D)), const((1, D)),
            const((D, D)), const((1, D)),
            const((4 * NPF, D)), const((1, D)), const((1, D)),
            const((D, D)), const((1, D)),
            const((D, D)), const((1, D)),
            const((1, NPF)),
        ],
        out_specs=[
            pl.BlockSpec((1, T, D), lambda b, i: (b, i, 0)),
            pl.BlockSpec((1, T, N), lambda b, i: (b, i, 0)),
        ],
        out_shape=[
            jax.ShapeDtypeStruct((bs, N, D), jnp.float32),
            jax.ShapeDtypeStruct((bs, N, N), jnp.bool_),
        ],
    )(pred_boxes, pred_boxes, boxesT, seed_mask, seed_row, tgt,
      W1.T, row(b1), W2.T, row(b2), w3wt, s3, row(b3),
      W4.T, row(b4), W5.T, row(b5), ang)
    return out, mask


# speedup vs baseline: 2.8316x; 1.1750x over previous
import numpy as np
import jax
import jax.numpy as jnp
from jax.experimental import pallas as pl

IOU_THR = 0.5
TOP_K = 10
NPF = 128


def _body(boxes_i_ref, boxes_ref, boxesT_ref, seed_col_ref, seed_row_ref,
          tgt_ref, w1t_ref, b1_ref, w2t_ref, b2_ref, w3wt_ref, s3_ref,
          b3_ref, w4t_ref, b4_ref, w5t_ref, b5_ref, ang_ref,
          out_ref, mask_ref):
    f32 = jnp.float32
    T = boxes_i_ref.shape[1]
    N = boxesT_ref.shape[2]
    D = tgt_ref.shape[2]

    bi = boxes_i_ref[0]
    bT = boxesT_ref[0]

    cx_i, cy_i, w_i, h_i = bi[:, 0:1], bi[:, 1:2], bi[:, 2:3], bi[:, 3:4]
    cx_j, cy_j, w_j, h_j = bT[0:1, :], bT[1:2, :], bT[2:3, :], bT[3:4, :]

    x0_i = cx_i - 0.5 * w_i
    x1_i = cx_i + 0.5 * w_i
    y0_i = cy_i - 0.5 * h_i
    y1_i = cy_i + 0.5 * h_i
    x0_j = cx_j - 0.5 * w_j
    x1_j = cx_j + 0.5 * w_j
    y0_j = cy_j - 0.5 * h_j
    y1_j = cy_j + 0.5 * h_j

    iw = jnp.maximum(jnp.minimum(x1_i, x1_j) - jnp.maximum(x0_i, x0_j), 0.0)
    ih = jnp.maximum(jnp.minimum(y1_i, y1_j) - jnp.maximum(y0_i, y0_j), 0.0)
    inter = iw * ih
    area_i = (x1_i - x0_i) * (y1_i - y0_i)
    area_j = (x1_j - x0_j) * (y1_j - y0_j)
    union = area_i + area_j - inter
    iou = inter / jnp.maximum(union, 1e-9)

    mask_ref[0] = iou >= IOU_THR

    neg_col = 1.0 - seed_col_ref[0]
    ov = iou * seed_row_ref[0] * neg_col

    iota = jax.lax.broadcasted_iota(jnp.int32, (T, N), 1)
    ang = ang_ref[...]
    boxes_all = boxes_ref[0]

    macc = jnp.full((T, D), -jnp.inf, dtype=f32)
    for _ in range(TOP_K):
        m = jnp.max(ov, axis=1, keepdims=True)
        idx = jnp.min(jnp.where(ov == m, iota, N), axis=1, keepdims=True)
        onehot = iota == idx
        nb = jnp.dot(onehot.astype(f32), boxes_all,
                     preferred_element_type=f32, precision=jax.lax.Precision.HIGHEST)
        ov = jnp.where(onehot, -1.0, ov)
        mk = (m >= IOU_THR).astype(f32)
        v = m * mk
        x = jnp.log(jnp.maximum(jnp.abs(nb - bi), 1e-7))
        waves = jnp.concatenate(
            [jnp.sin(x[:, 0:1] * ang), jnp.cos(x[:, 1:2] * ang),
             jnp.sin(x[:, 2:3] * ang), jnp.cos(x[:, 3:4] * ang)], axis=1)
        h = jnp.maximum(
            jnp.dot(waves, w3wt_ref[...], preferred_element_type=f32)
            + v * s3_ref[...] + b3_ref[...], 0.0)
        fk = jnp.dot(h, w4t_ref[...], preferred_element_type=f32) + b4_ref[...]
        macc = jnp.maximum(macc, fk * mk)

    h1 = jnp.maximum(
        jnp.dot(tgt_ref[0], w1t_ref[...], preferred_element_type=f32)
        + b1_ref[...], 0.0)
    cur = jnp.dot(h1, w2t_ref[...], preferred_element_type=f32) + b2_ref[...]
    pre = cur * neg_col + macc
    out = jnp.maximum(
        jnp.dot(pre, w5t_ref[...], preferred_element_type=f32)
        + b5_ref[...], 0.0) * neg_col
    out_ref[0] = out


def kernel(tgt, seed_mask, pred_boxes, W1, b1, W2, b2, W3, b3, W4, b4, W5, b5):
    bs, N, D = tgt.shape
    T = 1000

    boxesT = jnp.transpose(pred_boxes, (0, 2, 1))
    seed_row = jnp.transpose(seed_mask, (0, 2, 1))

    s3 = jnp.sum(W3[:, :64], axis=1)[None, :]
    w3wt = jnp.transpose(W3[:, 64:])

    dim_t = 10000.0 ** (2.0 * np.floor(np.arange(NPF) / 2.0) / NPF)
    ang = jnp.asarray((2.0 * np.pi) / dim_t, jnp.float32)[None, :]

    row = lambda b: b[None, :]

    def const(shape):
        return pl.BlockSpec(shape, lambda b, i: (0,) * len(shape))

    out, mask = pl.pallas_call(
        _body,
        grid=(bs, N // T),
        in_specs=[
            pl.BlockSpec((1, T, 4), lambda b, i: (b, i, 0)),
            pl.BlockSpec((1, N, 4), lambda b, i: (b, 0, 0)),
            pl.BlockSpec((1, 4, N), lambda b, i: (b, 0, 0)),
            pl.BlockSpec((1, T, 1), lambda b, i: (b, i, 0)),
            pl.BlockSpec((1, 1, N), lambda b, i: (b, 0, 0)),
            pl.BlockSpec((1, T, D), lambda b, i: (b, i, 0)),
            const((D, D)), const((1, D)),
            const((D, D)), const((1, D)),
            const((4 * NPF, D)), const((1, D)), const((1, D)),
            const((D, D)), const((1, D)),
            const((D, D)), const((1, D)),
            const((1, NPF)),
        ],
        out_specs=[
            pl.BlockSpec((1, T, D), lambda b, i: (b, i, 0)),
            pl.BlockSpec((1, T, N), lambda b, i: (b, i, 0)),
        ],
        out_shape=[
            jax.ShapeDtypeStruct((bs, N, D), jnp.float32),
            jax.ShapeDtypeStruct((bs, N, N), jnp.bool_),
        ],
    )(pred_boxes, pred_boxes, boxesT, seed_mask, seed_row, tgt,
      W1.T, row(b1), W2.T, row(b2), w3wt, s3, row(b3),
      W4.T, row(b4), W5.T, row(b5), ang)
    return out, mask


# speedup vs baseline: 4.2458x; 1.4995x over previous
import numpy as np
import jax
import jax.numpy as jnp
from jax.experimental import pallas as pl

IOU_THR = 0.5
TOP_K = 10
NPF = 128


def _body(boxes_i_ref, gh_ref, gm_ref, gl_ref, boxesT_ref, seed_col_ref,
          seed_row_ref, tgt_ref, w1t_ref, b1_ref, w2t_ref, b2_ref,
          w3wt_ref, s3_ref, b3_ref, w4t_ref, b4_ref, w5t_ref, b5_ref,
          ang_ref, out_ref, mask_ref):
    f32 = jnp.float32
    T = boxes_i_ref.shape[1]
    N = boxesT_ref.shape[2]
    D = tgt_ref.shape[2]

    bi = boxes_i_ref[0]
    bT = boxesT_ref[0]

    cx_i, cy_i, w_i, h_i = bi[:, 0:1], bi[:, 1:2], bi[:, 2:3], bi[:, 3:4]
    cx_j, cy_j, w_j, h_j = bT[0:1, :], bT[1:2, :], bT[2:3, :], bT[3:4, :]

    x0_i = cx_i - 0.5 * w_i
    x1_i = cx_i + 0.5 * w_i
    y0_i = cy_i - 0.5 * h_i
    y1_i = cy_i + 0.5 * h_i
    x0_j = cx_j - 0.5 * w_j
    x1_j = cx_j + 0.5 * w_j
    y0_j = cy_j - 0.5 * h_j
    y1_j = cy_j + 0.5 * h_j

    iw = jnp.maximum(jnp.minimum(x1_i, x1_j) - jnp.maximum(x0_i, x0_j), 0.0)
    ih = jnp.maximum(jnp.minimum(y1_i, y1_j) - jnp.maximum(y0_i, y0_j), 0.0)
    inter = iw * ih
    area_i = (x1_i - x0_i) * (y1_i - y0_i)
    area_j = (x1_j - x0_j) * (y1_j - y0_j)
    union = area_i + area_j - inter
    iou = inter / jnp.maximum(union, 1e-9)

    mask_ref[0] = iou >= IOU_THR

    neg_col = 1.0 - seed_col_ref[0]
    ov = iou * seed_row_ref[0] * neg_col

    iota = jax.lax.broadcasted_iota(jnp.int32, (T, N), 1)
    iota_s = jax.lax.broadcasted_iota(jnp.int32, (T, 256), 1)
    ang = ang_ref[...]
    gh = gh_ref[0]
    gm = gm_ref[0]
    gl = gl_ref[0]

    macc = jnp.full((T, D), -jnp.inf, dtype=f32)
    for _ in range(TOP_K):
        m = jnp.max(ov, axis=1, keepdims=True)
        idx = jnp.min(jnp.where(ov == m, iota, N), axis=1, keepdims=True)
        onehot = iota == idx
        r = jnp.bitwise_and(idx, 255)
        q = jnp.right_shift(idx, 8)
        oh = (iota_s == r).astype(jnp.bfloat16)
        nb4 = (jnp.dot(oh, gh, preferred_element_type=f32)
               + jnp.dot(oh, gm, preferred_element_type=f32)
               + jnp.dot(oh, gl, preferred_element_type=f32))
        nb = sum((q == g).astype(f32) * nb4[:, 4 * g:4 * g + 4]
                 for g in range(4))
        ov = jnp.where(onehot, -1.0, ov)
        mk = (m >= IOU_THR).astype(f32)
        v = m * mk
        x = jnp.log(jnp.maximum(jnp.abs(nb - bi), 1e-7))
        waves = jnp.concatenate(
            [jnp.sin(x[:, 0:1] * ang), jnp.cos(x[:, 1:2] * ang),
             jnp.sin(x[:, 2:3] * ang), jnp.cos(x[:, 3:4] * ang)], axis=1)
        h = jnp.maximum(
            jnp.dot(waves, w3wt_ref[...], preferred_element_type=f32)
            + v * s3_ref[...] + b3_ref[...], 0.0)
        fk = jnp.dot(h, w4t_ref[...], preferred_element_type=f32) + b4_ref[...]
        macc = jnp.maximum(macc, fk * mk)

    h1 = jnp.maximum(
        jnp.dot(tgt_ref[0], w1t_ref[...], preferred_element_type=f32)
        + b1_ref[...], 0.0)
    cur = jnp.dot(h1, w2t_ref[...], preferred_element_type=f32) + b2_ref[...]
    pre = cur * neg_col + macc
    out = jnp.maximum(
        jnp.dot(pre, w5t_ref[...], preferred_element_type=f32)
        + b5_ref[...], 0.0) * neg_col
    out_ref[0] = out


def kernel(tgt, seed_mask, pred_boxes, W1, b1, W2, b2, W3, b3, W4, b4, W5, b5):
    bs, N, D = tgt.shape
    T = 1000

    boxesT = jnp.transpose(pred_boxes, (0, 2, 1))
    seed_row = jnp.transpose(seed_mask, (0, 2, 1))

    f32 = jnp.float32
    bp = jnp.pad(pred_boxes, ((0, 0), (0, 1024 - N), (0, 0)))
    grouped = jnp.transpose(bp.reshape(bs, 4, 256, 4),
                            (0, 2, 1, 3)).reshape(bs, 256, 16)
    g_hi = grouped.astype(jnp.bfloat16)
    r1 = grouped - g_hi.astype(f32)
    g_mid = r1.astype(jnp.bfloat16)
    g_lo = (r1 - g_mid.astype(f32)).astype(jnp.bfloat16)

    s3 = jnp.sum(W3[:, :64], axis=1)[None, :]
    w3wt = jnp.transpose(W3[:, 64:])

    dim_t = 10000.0 ** (2.0 * np.floor(np.arange(NPF) / 2.0) / NPF)
    ang = jnp.asarray((2.0 * np.pi) / dim_t, jnp.float32)[None, :]

    row = lambda b: b[None, :]

    def const(shape):
        return pl.BlockSpec(shape, lambda b, i: (0,) * len(shape))

    out, mask = pl.pallas_call(
        _body,
        grid=(bs, N // T),
        in_specs=[
            pl.BlockSpec((1, T, 4), lambda b, i: (b, i, 0)),
            pl.BlockSpec((1, 256, 16), lambda b, i: (b, 0, 0)),
            pl.BlockSpec((1, 256, 16), lambda b, i: (b, 0, 0)),
            pl.BlockSpec((1, 256, 16), lambda b, i: (b, 0, 0)),
            pl.BlockSpec((1, 4, N), lambda b, i: (b, 0, 0)),
            pl.BlockSpec((1, T, 1), lambda b, i: (b, i, 0)),
            pl.BlockSpec((1, 1, N), lambda b, i: (b, 0, 0)),
            pl.BlockSpec((1, T, D), lambda b, i: (b, i, 0)),
            const((D, D)), const((1, D)),
            const((D, D)), const((1, D)),
            const((4 * NPF, D)), const((1, D)), const((1, D)),
            const((D, D)), const((1, D)),
            const((D, D)), const((1, D)),
            const((1, NPF)),
        ],
        out_specs=[
            pl.BlockSpec((1, T, D), lambda b, i: (b, i, 0)),
            pl.BlockSpec((1, T, N), lambda b, i: (b, i, 0)),
        ],
        out_shape=[
            jax.ShapeDtypeStruct((bs, N, D), jnp.float32),
            jax.ShapeDtypeStruct((bs, N, N), jnp.bool_),
        ],
    )(pred_boxes, g_hi, g_mid, g_lo, boxesT, seed_mask, seed_row, tgt,
      W1.T, row(b1), W2.T, row(b2), w3wt, s3, row(b3),
      W4.T, row(b4), W5.T, row(b5), ang)
    return out, mask
